# Initial kernel scaffold; baseline (speedup 1.0000x reference)
import jax
import jax.numpy as jnp
from jax.experimental import pallas as pl


def kernel(x, parameter_matrix):
    raise NotImplementedError("write your pallas kernel here")


# speedup vs baseline: 2.9353x; 2.9353x over previous
import functools

import jax
import jax.numpy as jnp
from jax import lax
from jax.experimental import pallas as pl
from jax.experimental.pallas import tpu as pltpu
from jax.experimental.pallas import tpu_sc as plsc

N = 4096
S = 1024
LANES = 16
CHUNKS = S // LANES
GROUP = 16


def _perm_xor(v, m):
    lane = jnp.arange(LANES, dtype=jnp.int32)
    return v.at[lane ^ m].get(mode="promise_in_bounds", unique_indices=True)


def _combine(x, y, m):
    lane = jnp.arange(LANES, dtype=jnp.int32)
    take_x = (lane & m) == 0
    t1 = jnp.where(take_x, x, y)
    t2 = jnp.where(take_x, y, x)
    return t1 + _perm_xor(t2, m)


def _lane_sums(accs):
    vecs = list(accs)
    m = 1
    while len(vecs) > 1:
        vecs = [_combine(vecs[2 * k], vecs[2 * k + 1], m)
                for k in range(len(vecs) // 2)]
        m *= 2
    return vecs[0]


def _dot_rows(w_vmem, vin_vmem, vout_vmem, nrows):
    def group_body(g, _):
        r0 = g * GROUP
        accs = [jnp.zeros((LANES,), jnp.float32) for _ in range(GROUP)]
        for c in range(CHUNKS):
            vc = vin_vmem[pl.ds(c * LANES, LANES)]
            for r in range(GROUP):
                accs[r] = accs[r] + w_vmem[r0 + r, pl.ds(c * LANES, LANES)] * vc
        vout_vmem[pl.ds(r0, LANES)] = _lane_sums(accs)
        return 0

    lax.fori_loop(0, nrows // GROUP, group_body, 0)


def _bias_act(vout_vmem, b_vmem, nrows, apply_silu):
    for k in range(nrows // LANES):
        sl = pl.ds(k * LANES, LANES)
        a = vout_vmem[sl] + b_vmem[sl]
        if apply_silu:
            a = a / (1.0 + jnp.exp(-a))
        vout_vmem[sl] = a


def _mlp_body(pm_hbm, x_hbm, b_hbm, out_hbm,
              w_vmem, vin_vmem, vout_vmem, b_vmem, shared1, shared2):
    cid = lax.axis_index("c")
    sid = lax.axis_index("s")

    pltpu.sync_copy(x_hbm, vin_vmem)
    pltpu.sync_copy(pm_hbm.at[pl.ds(S + sid * 64, 64), pl.ds(0, S)], w_vmem)
    pltpu.sync_copy(b_hbm.at[pl.ds(sid * 64, 64)], b_vmem)
    _dot_rows(w_vmem, vin_vmem, vout_vmem, 64)
    _bias_act(vout_vmem, b_vmem, 64, apply_silu=True)
    pltpu.sync_copy(vout_vmem, shared1.at[pl.ds(sid * 64, 64)])
    plsc.subcore_barrier()
    pltpu.sync_copy(shared1, vin_vmem)

    pltpu.sync_copy(pm_hbm.at[pl.ds(2 * S + sid * 64, 64), pl.ds(S, S)], w_vmem)
    pltpu.sync_copy(b_hbm.at[pl.ds(S + sid * 64, 64)], b_vmem)
    _dot_rows(w_vmem, vin_vmem, vout_vmem, 64)
    _bias_act(vout_vmem, b_vmem, 64, apply_silu=True)
    pltpu.sync_copy(vout_vmem, shared2.at[pl.ds(sid * 64, 64)])
    plsc.subcore_barrier()
    pltpu.sync_copy(shared2, vin_vmem)

    out0 = cid * 512 + sid * 32
    pltpu.sync_copy(pm_hbm.at[pl.ds(3 * S + out0, 32), pl.ds(2 * S, S)],
                    w_vmem.at[pl.ds(0, 32)])
    pltpu.sync_copy(b_hbm.at[pl.ds(2 * S + out0, 32)], b_vmem.at[pl.ds(0, 32)])
    _dot_rows(w_vmem, vin_vmem, vout_vmem, 32)
    _bias_act(vout_vmem, b_vmem, 32, apply_silu=False)
    pltpu.sync_copy(vout_vmem.at[pl.ds(0, 32)], out_hbm.at[pl.ds(out0, 32)])


def kernel(x, parameter_matrix):
    b_all = parameter_matrix[S:, N]

    mesh = plsc.VectorSubcoreMesh(core_axis_name="c", subcore_axis_name="s")
    k = functools.partial(
        pl.kernel,
        mesh=mesh,
        out_type=jax.ShapeDtypeStruct((S,), jnp.float32),
        scratch_types=[
            pltpu.VMEM((64, S), jnp.float32),
            pltpu.VMEM((S,), jnp.float32),
            pltpu.VMEM((64,), jnp.float32),
            pltpu.VMEM((64,), jnp.float32),
            pltpu.VMEM_SHARED((S,), jnp.float32),
            pltpu.VMEM_SHARED((S,), jnp.float32),
        ],
    )(_mlp_body)
    return k(parameter_matrix, x, b_all)


# speedup vs baseline: 4.0526x; 1.3806x over previous
import functools

import jax
import jax.numpy as jnp
from jax import lax
from jax.experimental import pallas as pl
from jax.experimental.pallas import tpu as pltpu
from jax.experimental.pallas import tpu_sc as plsc

N = 4096
S = 1024
LANES = 16
CHUNKS = S // LANES
GROUP = 16


def _perm_xor(v, m):
    lane = jnp.arange(LANES, dtype=jnp.int32)
    return v.at[lane ^ m].get(mode="promise_in_bounds", unique_indices=True)


def _combine(x, y, m):
    lane = jnp.arange(LANES, dtype=jnp.int32)
    take_x = (lane & m) == 0
    t1 = jnp.where(take_x, x, y)
    t2 = jnp.where(take_x, y, x)
    return t1 + _perm_xor(t2, m)


def _lane_sums(accs):
    vecs = list(accs)
    m = 1
    while len(vecs) > 1:
        vecs = [_combine(vecs[2 * k], vecs[2 * k + 1], m)
                for k in range(len(vecs) // 2)]
        m *= 2
    return vecs[0]


def _dot_rows(w_vmem, vin_vmem, vout_vmem, nrows):
    def group_body(g, _):
        r0 = g * GROUP
        accs = [jnp.zeros((LANES,), jnp.float32) for _ in range(GROUP)]
        for c in range(CHUNKS):
            vc = vin_vmem[pl.ds(c * LANES, LANES)]
            for r in range(GROUP):
                accs[r] = accs[r] + w_vmem[r0 + r, pl.ds(c * LANES, LANES)] * vc
        vout_vmem[pl.ds(r0, LANES)] = _lane_sums(accs)
        return 0

    lax.fori_loop(0, nrows // GROUP, group_body, 0)


def _bias_act(vout_vmem, b_vmem, nrows, apply_silu):
    for k in range(nrows // LANES):
        sl = pl.ds(k * LANES, LANES)
        a = vout_vmem[sl] + b_vmem[sl]
        if apply_silu:
            a = a / (1.0 + jnp.exp(-a))
        vout_vmem[sl] = a


def _mlp_body(w1_hbm, w2_hbm, w3_hbm, x_hbm, b_hbm, out_hbm,
              w_vmem, vin_vmem, vout_vmem, b_vmem, shared1, shared2):
    cid = lax.axis_index("c")
    sid = lax.axis_index("s")

    pltpu.sync_copy(x_hbm, vin_vmem)
    pltpu.sync_copy(w1_hbm.at[pl.ds(sid * 64, 64)], w_vmem)
    pltpu.sync_copy(b_hbm.at[pl.ds(sid * 64, 64)], b_vmem)
    _dot_rows(w_vmem, vin_vmem, vout_vmem, 64)
    _bias_act(vout_vmem, b_vmem, 64, apply_silu=True)
    pltpu.sync_copy(vout_vmem, shared1.at[pl.ds(sid * 64, 64)])
    plsc.subcore_barrier()
    pltpu.sync_copy(shared1, vin_vmem)

    pltpu.sync_copy(w2_hbm.at[pl.ds(sid * 64, 64)], w_vmem)
    pltpu.sync_copy(b_hbm.at[pl.ds(S + sid * 64, 64)], b_vmem)
    _dot_rows(w_vmem, vin_vmem, vout_vmem, 64)
    _bias_act(vout_vmem, b_vmem, 64, apply_silu=True)
    pltpu.sync_copy(vout_vmem, shared2.at[pl.ds(sid * 64, 64)])
    plsc.subcore_barrier()
    pltpu.sync_copy(shared2, vin_vmem)

    out0 = cid * 512 + sid * 32
    pltpu.sync_copy(w3_hbm.at[pl.ds(out0, 32)], w_vmem.at[pl.ds(0, 32)])
    pltpu.sync_copy(b_hbm.at[pl.ds(2 * S + out0, 32)], b_vmem.at[pl.ds(0, 32)])
    _dot_rows(w_vmem, vin_vmem, vout_vmem, 32)
    _bias_act(vout_vmem, b_vmem, 32, apply_silu=False)
    pltpu.sync_copy(vout_vmem.at[pl.ds(0, 32)], out_hbm.at[pl.ds(out0, 32)])


def kernel(x, parameter_matrix):
    w1 = parameter_matrix[S:2 * S, 0:S]
    w2 = parameter_matrix[2 * S:3 * S, S:2 * S]
    w3 = parameter_matrix[3 * S:4 * S, 2 * S:3 * S]
    b_all = parameter_matrix[S:, N]

    mesh = plsc.VectorSubcoreMesh(core_axis_name="c", subcore_axis_name="s")
    k = functools.partial(
        pl.kernel,
        mesh=mesh,
        out_type=jax.ShapeDtypeStruct((S,), jnp.float32),
        scratch_types=[
            pltpu.VMEM((64, S), jnp.float32),
            pltpu.VMEM((S,), jnp.float32),
            pltpu.VMEM((64,), jnp.float32),
            pltpu.VMEM((64,), jnp.float32),
            pltpu.VMEM_SHARED((S,), jnp.float32),
            pltpu.VMEM_SHARED((S,), jnp.float32),
        ],
    )(_mlp_body)
    return k(w1, w2, w3, x, b_all)
